# baseline (speedup 1.0000x reference)
import functools

import jax
import jax.numpy as jnp
from jax import lax
from jax.experimental import pallas as pl
from jax.experimental.pallas import tpu as pltpu
from jax.experimental.pallas import tpu_sc as plsc

_N = 10000
_NP = 10240
_E = 320000
_D = 128
_NC = 2
_NS = 16
_CH = 128
_NCH = 80
_SCH = 128
_SNCH = 80
_NBUF = 2
_EP = _NC * _NS * _NCH * _CH
_ROWS_PER_TILE = _NP // _NS

_mesh = plsc.VectorSubcoreMesh(core_axis_name="c", subcore_axis_name="s")


def _fill(buf, rows, cols, value):
    @pl.loop(0, rows)
    def _(r):
        for j in range(cols // 16):
            buf.at[pl.ds(r, 1), pl.ds(j * 16, 16)][...] = jnp.full(
                (1, 16), value, jnp.float32)


def _zero_acc(zbuf, rows, cols, acc, base):
    _fill(zbuf, rows, cols, 0.0)
    for q in range(_ROWS_PER_TILE // rows):
        pltpu.sync_copy(zbuf, acc.at[pl.ds(base + q * rows, rows)])


def _sc_segsum(u, src3, dst3):
    @functools.partial(
        pl.kernel,
        out_type=jax.ShapeDtypeStruct((_NC, _NP, _D), jnp.float32),
        mesh=_mesh,
        scratch_types=[
            pltpu.VMEM((_SNCH // 4, _SCH), jnp.int32),
            pltpu.VMEM((_SNCH // 4, _SCH), jnp.int32),
            pltpu.VMEM((_NBUF, _SCH, _D), jnp.float32),
            pltpu.VMEM_SHARED((_NP, _D), jnp.float32),
            pltpu.SemaphoreType.DMA,
            pltpu.SemaphoreType.DMA,
            pltpu.SemaphoreType.DMA,
            pltpu.SemaphoreType.DMA,
        ],
    )
    def k(u_hbm, src_hbm, dst_hbm, p_hbm, si, di, ring, acc,
          sg0, sg1, ss0, ss1):
        sgs = (sg0, sg1)
        sss = (ss0, ss1)
        c = lax.axis_index("c")
        s = lax.axis_index("s")
        w = c * _NS + s
        base = s * _ROWS_PER_TILE
        _zero_acc(ring.at[0], _SCH, _D, acc, base)
        plsc.subcore_barrier()

        hch = _SNCH // 4
        for half in range(4):
            pltpu.sync_copy(src_hbm.at[w, half], si)
            pltpu.sync_copy(dst_hbm.at[w, half], di)

            @pl.loop(0, hch + _NBUF, step=_NBUF)
            def _(t0):
                for b in range(_NBUF):
                    t = t0 + b
                    b1 = (b + 1) % _NBUF
                    t1 = t - 1

                    @pl.when(jnp.logical_and(t1 >= 0, t1 < hch))
                    def _():
                        pltpu.make_async_copy(
                            u_hbm.at[si.at[0]], ring.at[b1],
                            sgs[b1]).wait()
                        pltpu.async_copy(ring.at[b1], acc.at[di.at[t1]],
                                         sss[b1], add=True)

                    @pl.when(jnp.logical_and(t >= _NBUF, t < hch + _NBUF))
                    def _():
                        pltpu.make_async_copy(
                            ring.at[b], acc.at[di.at[0]], sss[b]).wait()

                    @pl.when(t < hch)
                    def _():
                        pltpu.async_copy(u_hbm.at[si.at[t]], ring.at[b],
                                         sgs[b])

        plsc.subcore_barrier()
        pltpu.sync_copy(acc.at[pl.ds(base, _ROWS_PER_TILE)],
                        p_hbm.at[c].at[pl.ds(base, _ROWS_PER_TILE)])

    return k(u, src3, dst3)


def _sc_degree(dst3):
    @functools.partial(
        pl.kernel,
        out_type=jax.ShapeDtypeStruct((_NC, _NP, _D), jnp.float32),
        mesh=_mesh,
        scratch_types=[
            pltpu.VMEM((_NCH, _CH), jnp.int32),
            pltpu.VMEM((_CH, _D), jnp.float32),
            pltpu.VMEM_SHARED((_NP, _D), jnp.float32),
            pltpu.SemaphoreType.DMA,
        ],
    )
    def k(dst_hbm, deg_hbm, di, ones, acc, sem):
        c = lax.axis_index("c")
        s = lax.axis_index("s")
        w = c * _NS + s
        pltpu.sync_copy(dst_hbm.at[w], di)
        base = s * _ROWS_PER_TILE
        _zero_acc(ones, _CH, _D, acc, base)
        _fill(ones, _CH, _D, 1.0)
        plsc.subcore_barrier()

        @pl.loop(0, _NCH)
        def _(t):
            pltpu.sync_copy(ones, acc.at[di.at[t]], add=True)

        plsc.subcore_barrier()
        pltpu.sync_copy(acc.at[pl.ds(base, _ROWS_PER_TILE)],
                        deg_hbm.at[c].at[pl.ds(base, _ROWS_PER_TILE)])

    return k(dst3)


_R = 400
_G = _N // _R


def _tc_prep(x, degp, w0):
    def body(x_ref, deg_ref, w_ref, acc_ref, u_ref, dis_ref):
        deg = deg_ref[0][:, 0:1] + deg_ref[1][:, 0:1]
        dis = jnp.where(deg > 0, lax.rsqrt(jnp.maximum(deg, 1e-12)), 0.0)
        dis_ref[...] = dis
        xb = x_ref[...]
        acc_ref[...] = jnp.dot(xb, w_ref[...],
                               preferred_element_type=jnp.float32)
        u_ref[...] = xb * dis

    return pl.pallas_call(
        body,
        grid=(_G,),
        in_specs=[
            pl.BlockSpec((_R, _D), lambda i: (i, 0)),
            pl.BlockSpec((2, _R, _D), lambda i: (0, i, 0)),
            pl.BlockSpec((_D, _D), lambda i: (0, 0)),
        ],
        out_specs=[
            pl.BlockSpec((_R, _D), lambda i: (i, 0)),
            pl.BlockSpec((_R, _D), lambda i: (i, 0)),
            pl.BlockSpec((_R, 1), lambda i: (i, 0)),
        ],
        out_shape=[
            jax.ShapeDtypeStruct((_N, _D), jnp.float32),
            jax.ShapeDtypeStruct((_N, _D), jnp.float32),
            jax.ShapeDtypeStruct((_N, 1), jnp.float32),
        ],
    )(x, degp, w0)


def _tc_hop(p2, dis, w, acc):
    def body(p_ref, dis_ref, w_ref, acc_ref, out_ref, u_ref):
        d = dis_ref[...]
        h = (p_ref[0] + p_ref[1]) * d
        out_ref[...] = acc_ref[...] + jnp.dot(
            h, w_ref[...], preferred_element_type=jnp.float32)
        u_ref[...] = h * d

    return pl.pallas_call(
        body,
        grid=(_G,),
        in_specs=[
            pl.BlockSpec((2, _R, _D), lambda i: (0, i, 0)),
            pl.BlockSpec((_R, 1), lambda i: (i, 0)),
            pl.BlockSpec((_D, _D), lambda i: (0, 0)),
            pl.BlockSpec((_R, _D), lambda i: (i, 0)),
        ],
        out_specs=[
            pl.BlockSpec((_R, _D), lambda i: (i, 0)),
            pl.BlockSpec((_R, _D), lambda i: (i, 0)),
        ],
        out_shape=[
            jax.ShapeDtypeStruct((_N, _D), jnp.float32),
            jax.ShapeDtypeStruct((_N, _D), jnp.float32),
        ],
    )(p2, dis, w, acc)


def _tc_mid(p2, dis, w3, acc, b0, w10):
    def body(p_ref, dis_ref, w3_ref, acc_ref, b_ref, w10_ref,
             acc1_ref, u_ref):
        d = dis_ref[...]
        h = (p_ref[0] + p_ref[1]) * d
        o = (acc_ref[...]
             + jnp.dot(h, w3_ref[...], preferred_element_type=jnp.float32)
             + b_ref[...])
        t = jnp.maximum(o, 0.0)
        acc1_ref[...] = jnp.dot(t, w10_ref[...],
                                preferred_element_type=jnp.float32)
        u_ref[...] = t * d

    return pl.pallas_call(
        body,
        grid=(_G,),
        in_specs=[
            pl.BlockSpec((2, _R, _D), lambda i: (0, i, 0)),
            pl.BlockSpec((_R, 1), lambda i: (i, 0)),
            pl.BlockSpec((_D, _D), lambda i: (0, 0)),
            pl.BlockSpec((_R, _D), lambda i: (i, 0)),
            pl.BlockSpec((1, _D), lambda i: (0, 0)),
            pl.BlockSpec((_D, _D), lambda i: (0, 0)),
        ],
        out_specs=[
            pl.BlockSpec((_R, _D), lambda i: (i, 0)),
            pl.BlockSpec((_R, _D), lambda i: (i, 0)),
        ],
        out_shape=[
            jax.ShapeDtypeStruct((_N, _D), jnp.float32),
            jax.ShapeDtypeStruct((_N, _D), jnp.float32),
        ],
    )(p2, dis, w3, acc, b0, w10)


def _tc_last(p2, dis, w, acc, b1):
    def body(p_ref, dis_ref, w_ref, acc_ref, b_ref, out_ref):
        d = dis_ref[...]
        h = (p_ref[0] + p_ref[1]) * d
        out_ref[...] = (acc_ref[...]
                        + jnp.dot(h, w_ref[...],
                                  preferred_element_type=jnp.float32)
                        + b_ref[...])

    return pl.pallas_call(
        body,
        grid=(_G,),
        in_specs=[
            pl.BlockSpec((2, _R, _D), lambda i: (0, i, 0)),
            pl.BlockSpec((_R, 1), lambda i: (i, 0)),
            pl.BlockSpec((_D, _D), lambda i: (0, 0)),
            pl.BlockSpec((_R, _D), lambda i: (i, 0)),
            pl.BlockSpec((1, _D), lambda i: (0, 0)),
        ],
        out_specs=pl.BlockSpec((_R, _D), lambda i: (i, 0)),
        out_shape=jax.ShapeDtypeStruct((_N, _D), jnp.float32),
    )(p2, dis, w, acc, b1)


def kernel(x, edge_index, W0, b0, W1, b1):
    src = edge_index[0]
    dst = edge_index[1]
    npad = _EP - _E
    pad_src = (jnp.arange(npad, dtype=jnp.int32) * 131) % _N
    pad_dst = _N + (jnp.arange(npad, dtype=jnp.int32) % (_NP - _N))
    srcp = jnp.concatenate([src, pad_src])
    dstp = jnp.concatenate([dst, pad_dst])
    src3 = srcp.reshape(_NC * _NS, 4, _SNCH // 4, _SCH)
    dst3 = dstp.reshape(_NC * _NS, 4, _SNCH // 4, _SCH)
    dst3d = dstp.reshape(_NC * _NS, _NCH, _CH)

    degp = _sc_degree(dst3d)
    acc, u, dis = _tc_prep(x, degp, W0[0])
    for k in (1, 2):
        p = _sc_segsum(u, src3, dst3)
        acc, u = _tc_hop(p, dis, W0[k], acc)
    p = _sc_segsum(u, src3, dst3)
    acc1, u = _tc_mid(p, dis, W0[3], acc, b0.reshape(1, _D), W1[0])
    for k in (1, 2):
        p = _sc_segsum(u, src3, dst3)
        acc1, u = _tc_hop(p, dis, W1[k], acc1)
    p = _sc_segsum(u, src3, dst3)
    return _tc_last(p, dis, W1[3], acc1, b1.reshape(1, _D))


# speedup vs baseline: 1.1375x; 1.1375x over previous
import functools

import jax
import jax.numpy as jnp
from jax import lax
from jax.experimental import pallas as pl
from jax.experimental.pallas import tpu as pltpu
from jax.experimental.pallas import tpu_sc as plsc

_N = 10000
_NP = 10240
_E = 320000
_D = 128
_NC = 2
_NS = 16
_CH = 128
_NCH = 80
_SCH = 128
_SNCH = 80
_NBUF = 2
_EP = _NC * _NS * _NCH * _CH
_ROWS_PER_TILE = _NP // _NS

_mesh = plsc.VectorSubcoreMesh(core_axis_name="c", subcore_axis_name="s")


def _fill(buf, rows, cols, value):
    @pl.loop(0, rows)
    def _(r):
        for j in range(cols // 16):
            buf.at[pl.ds(r, 1), pl.ds(j * 16, 16)][...] = jnp.full(
                (1, 16), value, jnp.float32)


def _zero_acc(zbuf, rows, cols, acc, base):
    _fill(zbuf, rows, cols, 0.0)
    for q in range(_ROWS_PER_TILE // rows):
        pltpu.sync_copy(zbuf, acc.at[pl.ds(base + q * rows, rows)])


def _sc_segsum(u, src3, dst3):
    @functools.partial(
        pl.kernel,
        out_type=jax.ShapeDtypeStruct((_NC, _NP, _D), jnp.float32),
        mesh=_mesh,
        scratch_types=[
            pltpu.VMEM((_SNCH // 4, _SCH), jnp.int32),
            pltpu.VMEM((_SNCH // 4, _SCH), jnp.int32),
            pltpu.VMEM((_NBUF, _SCH, _D), jnp.float32),
            pltpu.VMEM_SHARED((_NP, _D), jnp.float32),
            pltpu.SemaphoreType.DMA,
            pltpu.SemaphoreType.DMA,
            pltpu.SemaphoreType.DMA,
            pltpu.SemaphoreType.DMA,
        ],
    )
    def k(u_hbm, src_hbm, dst_hbm, p_hbm, si, di, ring, acc,
          sg0, sg1, ss0, ss1):
        sgs = (sg0, sg1)
        sss = (ss0, ss1)
        c = lax.axis_index("c")
        s = lax.axis_index("s")
        w = c * _NS + s
        base = s * _ROWS_PER_TILE
        _zero_acc(ring.at[0], _SCH, _D, acc, base)
        plsc.subcore_barrier()

        hch = _SNCH // 4
        for half in range(4):
            pltpu.sync_copy(src_hbm.at[w, half], si)
            pltpu.sync_copy(dst_hbm.at[w, half], di)

            @pl.loop(0, hch + _NBUF, step=_NBUF)
            def _(t0):
                for b in range(_NBUF):
                    t = t0 + b
                    b1 = (b + 1) % _NBUF

                    @pl.when(jnp.logical_and(t >= _NBUF, t < hch + _NBUF))
                    def _():
                        pltpu.make_async_copy(
                            ring.at[b], acc.at[di.at[0]], sss[b]).wait()

                    @pl.when(t < hch)
                    def _():
                        pltpu.async_copy(u_hbm.at[si.at[t]], ring.at[b],
                                         sgs[b])

                    t1 = t - 1

                    @pl.when(jnp.logical_and(t1 >= 0, t1 < hch))
                    def _():
                        pltpu.make_async_copy(
                            u_hbm.at[si.at[0]], ring.at[b1],
                            sgs[b1]).wait()
                        pltpu.async_copy(ring.at[b1], acc.at[di.at[t1]],
                                         sss[b1], add=True)

        plsc.subcore_barrier()
        pltpu.sync_copy(acc.at[pl.ds(base, _ROWS_PER_TILE)],
                        p_hbm.at[c].at[pl.ds(base, _ROWS_PER_TILE)])

    return k(u, src3, dst3)


def _sc_degree(dst3):
    @functools.partial(
        pl.kernel,
        out_type=jax.ShapeDtypeStruct((_NC, _NP, _D), jnp.float32),
        mesh=_mesh,
        scratch_types=[
            pltpu.VMEM((_NCH, _CH), jnp.int32),
            pltpu.VMEM((_CH, _D), jnp.float32),
            pltpu.VMEM_SHARED((_NP, _D), jnp.float32),
            pltpu.SemaphoreType.DMA,
        ],
    )
    def k(dst_hbm, deg_hbm, di, ones, acc, sem):
        c = lax.axis_index("c")
        s = lax.axis_index("s")
        w = c * _NS + s
        pltpu.sync_copy(dst_hbm.at[w], di)
        base = s * _ROWS_PER_TILE
        _zero_acc(ones, _CH, _D, acc, base)
        _fill(ones, _CH, _D, 1.0)
        plsc.subcore_barrier()

        @pl.loop(0, _NCH)
        def _(t):
            pltpu.sync_copy(ones, acc.at[di.at[t]], add=True)

        plsc.subcore_barrier()
        pltpu.sync_copy(acc.at[pl.ds(base, _ROWS_PER_TILE)],
                        deg_hbm.at[c].at[pl.ds(base, _ROWS_PER_TILE)])

    return k(dst3)


_R = 400
_G = _N // _R


def _tc_matmul(x, w):
    def body(x_ref, w_ref, acc_ref):
        acc_ref[...] = jnp.dot(x_ref[...], w_ref[...],
                               preferred_element_type=jnp.float32)

    return pl.pallas_call(
        body,
        grid=(_G,),
        in_specs=[
            pl.BlockSpec((_R, _D), lambda i: (i, 0)),
            pl.BlockSpec((_D, _D), lambda i: (0, 0)),
        ],
        out_specs=pl.BlockSpec((_R, _D), lambda i: (i, 0)),
        out_shape=jax.ShapeDtypeStruct((_N, _D), jnp.float32),
    )(x, w)


def _tc_prep(x, degp):
    def body(x_ref, deg_ref, u_ref, dis_ref):
        deg = deg_ref[0][:, 0:1] + deg_ref[1][:, 0:1]
        dis = jnp.where(deg > 0, lax.rsqrt(jnp.maximum(deg, 1e-12)), 0.0)
        dis_ref[...] = dis
        u_ref[...] = x_ref[...] * dis

    return pl.pallas_call(
        body,
        grid=(_G,),
        in_specs=[
            pl.BlockSpec((_R, _D), lambda i: (i, 0)),
            pl.BlockSpec((2, _R, _D), lambda i: (0, i, 0)),
        ],
        out_specs=[
            pl.BlockSpec((_R, _D), lambda i: (i, 0)),
            pl.BlockSpec((_R, 1), lambda i: (i, 0)),
        ],
        out_shape=[
            jax.ShapeDtypeStruct((_N, _D), jnp.float32),
            jax.ShapeDtypeStruct((_N, 1), jnp.float32),
        ],
    )(x, degp)


def _tc_u(p2, dis):
    def body(p_ref, dis_ref, u_ref):
        d = dis_ref[...]
        u_ref[...] = (p_ref[0] + p_ref[1]) * (d * d)

    return pl.pallas_call(
        body,
        grid=(_G,),
        in_specs=[
            pl.BlockSpec((2, _R, _D), lambda i: (0, i, 0)),
            pl.BlockSpec((_R, 1), lambda i: (i, 0)),
        ],
        out_specs=pl.BlockSpec((_R, _D), lambda i: (i, 0)),
        out_shape=jax.ShapeDtypeStruct((_N, _D), jnp.float32),
    )(p2, dis)


def _tc_acc(p2, dis, w, acc):
    def body(p_ref, dis_ref, w_ref, acc_ref, out_ref):
        d = dis_ref[...]
        h = (p_ref[0] + p_ref[1]) * d
        out_ref[...] = acc_ref[...] + jnp.dot(
            h, w_ref[...], preferred_element_type=jnp.float32)

    return pl.pallas_call(
        body,
        grid=(_G,),
        in_specs=[
            pl.BlockSpec((2, _R, _D), lambda i: (0, i, 0)),
            pl.BlockSpec((_R, 1), lambda i: (i, 0)),
            pl.BlockSpec((_D, _D), lambda i: (0, 0)),
            pl.BlockSpec((_R, _D), lambda i: (i, 0)),
        ],
        out_specs=pl.BlockSpec((_R, _D), lambda i: (i, 0)),
        out_shape=jax.ShapeDtypeStruct((_N, _D), jnp.float32),
    )(p2, dis, w, acc)


def _tc_mid(p2, dis, w3, acc, b0, w10):
    def body(p_ref, dis_ref, w3_ref, acc_ref, b_ref, w10_ref,
             acc1_ref, u_ref):
        d = dis_ref[...]
        h = (p_ref[0] + p_ref[1]) * d
        o = (acc_ref[...]
             + jnp.dot(h, w3_ref[...], preferred_element_type=jnp.float32)
             + b_ref[...])
        t = jnp.maximum(o, 0.0)
        acc1_ref[...] = jnp.dot(t, w10_ref[...],
                                preferred_element_type=jnp.float32)
        u_ref[...] = t * d

    return pl.pallas_call(
        body,
        grid=(_G,),
        in_specs=[
            pl.BlockSpec((2, _R, _D), lambda i: (0, i, 0)),
            pl.BlockSpec((_R, 1), lambda i: (i, 0)),
            pl.BlockSpec((_D, _D), lambda i: (0, 0)),
            pl.BlockSpec((_R, _D), lambda i: (i, 0)),
            pl.BlockSpec((1, _D), lambda i: (0, 0)),
            pl.BlockSpec((_D, _D), lambda i: (0, 0)),
        ],
        out_specs=[
            pl.BlockSpec((_R, _D), lambda i: (i, 0)),
            pl.BlockSpec((_R, _D), lambda i: (i, 0)),
        ],
        out_shape=[
            jax.ShapeDtypeStruct((_N, _D), jnp.float32),
            jax.ShapeDtypeStruct((_N, _D), jnp.float32),
        ],
    )(p2, dis, w3, acc, b0, w10)


def _tc_last(p2, dis, w, acc, b1):
    def body(p_ref, dis_ref, w_ref, acc_ref, b_ref, out_ref):
        d = dis_ref[...]
        h = (p_ref[0] + p_ref[1]) * d
        out_ref[...] = (acc_ref[...]
                        + jnp.dot(h, w_ref[...],
                                  preferred_element_type=jnp.float32)
                        + b_ref[...])

    return pl.pallas_call(
        body,
        grid=(_G,),
        in_specs=[
            pl.BlockSpec((2, _R, _D), lambda i: (0, i, 0)),
            pl.BlockSpec((_R, 1), lambda i: (i, 0)),
            pl.BlockSpec((_D, _D), lambda i: (0, 0)),
            pl.BlockSpec((_R, _D), lambda i: (i, 0)),
            pl.BlockSpec((1, _D), lambda i: (0, 0)),
        ],
        out_specs=pl.BlockSpec((_R, _D), lambda i: (i, 0)),
        out_shape=jax.ShapeDtypeStruct((_N, _D), jnp.float32),
    )(p2, dis, w, acc, b1)


def kernel(x, edge_index, W0, b0, W1, b1):
    src = edge_index[0]
    dst = edge_index[1]
    npad = _EP - _E
    pad_src = (jnp.arange(npad, dtype=jnp.int32) * 131) % _N
    pad_dst = _N + (jnp.arange(npad, dtype=jnp.int32) % (_NP - _N))
    srcp = jnp.concatenate([src, pad_src])
    dstp = jnp.concatenate([dst, pad_dst])
    src3 = srcp.reshape(_NC * _NS, 4, _SNCH // 4, _SCH)
    dst3 = dstp.reshape(_NC * _NS, 4, _SNCH // 4, _SCH)
    dst3d = dstp.reshape(_NC * _NS, _NCH, _CH)

    degp = _sc_degree(dst3d)
    acc = _tc_matmul(x, W0[0])
    u, dis = _tc_prep(x, degp)
    for k in (1, 2):
        p = _sc_segsum(u, src3, dst3)
        u = _tc_u(p, dis)
        acc = _tc_acc(p, dis, W0[k], acc)
    p = _sc_segsum(u, src3, dst3)
    acc1, u = _tc_mid(p, dis, W0[3], acc, b0.reshape(1, _D), W1[0])
    for k in (1, 2):
        p = _sc_segsum(u, src3, dst3)
        u = _tc_u(p, dis)
        acc1 = _tc_acc(p, dis, W1[k], acc1)
    p = _sc_segsum(u, src3, dst3)
    return _tc_last(p, dis, W1[3], acc1, b1.reshape(1, _D))


# speedup vs baseline: 1.2028x; 1.0573x over previous
import functools

import jax
import jax.numpy as jnp
from jax import lax
from jax.experimental import pallas as pl
from jax.experimental.pallas import tpu as pltpu
from jax.experimental.pallas import tpu_sc as plsc

_N = 10000
_NP = 10240
_E = 320000
_D = 128
_NC = 2
_NS = 16
_CH = 128
_NCH = 80
_SCH = 128
_SNCH = 80
_SEC = 10
_NBUF = 2
_EP = _NC * _NS * _NCH * _CH
_ROWS_PER_TILE = _NP // _NS

_mesh = plsc.VectorSubcoreMesh(core_axis_name="c", subcore_axis_name="s")


def _fill(buf, rows, cols, value):
    @pl.loop(0, rows)
    def _(r):
        for j in range(cols // 16):
            buf.at[pl.ds(r, 1), pl.ds(j * 16, 16)][...] = jnp.full(
                (1, 16), value, jnp.float32)


def _zero_acc(zbuf, rows, cols, acc, base):
    _fill(zbuf, rows, cols, 0.0)
    for q in range(_ROWS_PER_TILE // rows):
        pltpu.sync_copy(zbuf, acc.at[pl.ds(base + q * rows, rows)])


def _sc_segsum(u, src3, dst3):
    @functools.partial(
        pl.kernel,
        out_type=jax.ShapeDtypeStruct((_NC, _NP, _D), jnp.float32),
        mesh=_mesh,
        scratch_types=[
            pltpu.VMEM((2, _SEC, _SCH), jnp.int32),
            pltpu.VMEM((2, _SEC, _SCH), jnp.int32),
            pltpu.VMEM((_NBUF, _SCH, _D), jnp.float32),
            pltpu.VMEM_SHARED((_NP, _D), jnp.float32),
            pltpu.SemaphoreType.DMA,
            pltpu.SemaphoreType.DMA,
            pltpu.SemaphoreType.DMA,
            pltpu.SemaphoreType.DMA,
            pltpu.SemaphoreType.DMA,
            pltpu.SemaphoreType.DMA,
        ],
    )
    def k(u_hbm, src_hbm, dst_hbm, p_hbm, si, di, ring, acc,
          sg0, sg1, ss0, ss1, sxs, sxd):
        sgs = (sg0, sg1)
        sss = (ss0, ss1)
        c = lax.axis_index("c")
        s = lax.axis_index("s")
        w = c * _NS + s
        base = s * _ROWS_PER_TILE
        nsec = _SNCH // _SEC
        ld0 = pltpu.async_copy(src_hbm.at[w, 0], si.at[0], sxs)
        ld1 = pltpu.async_copy(dst_hbm.at[w, 0], di.at[0], sxd)
        _zero_acc(ring.at[0], _SCH, _D, acc, base)
        ld0.wait()
        ld1.wait()
        plsc.subcore_barrier()

        gd = [None] * _SNCH
        sd = [None] * _SNCH
        for t in range(_SNCH):
            q, r = divmod(t, _SEC)
            b = t % _NBUF
            if r == 2 and q + 1 < nsec:
                qb = (q + 1) % 2
                pltpu.async_copy(src_hbm.at[w, q + 1], si.at[qb], sxs)
                pltpu.async_copy(dst_hbm.at[w, q + 1], di.at[qb], sxd)
            if r == 0 and q > 0:
                pltpu.make_async_copy(src_hbm.at[w, 0], si.at[q % 2],
                                      sxs).wait()
                pltpu.make_async_copy(dst_hbm.at[w, 0], di.at[q % 2],
                                      sxd).wait()
            if t >= _NBUF:
                sd[t - _NBUF].wait()
            gd[t] = pltpu.async_copy(u_hbm.at[si.at[q % 2, r]], ring.at[b],
                                     sgs[b])
            if t >= 1:
                t1 = t - 1
                q1, r1 = divmod(t1, _SEC)
                gd[t1].wait()
                sd[t1] = pltpu.async_copy(ring.at[t1 % _NBUF],
                                          acc.at[di.at[q1 % 2, r1]],
                                          sss[t1 % _NBUF], add=True)
        gd[_SNCH - 1].wait()
        sd[_SNCH - 1] = pltpu.async_copy(
            ring.at[(_SNCH - 1) % _NBUF],
            acc.at[di.at[(nsec - 1) % 2, _SEC - 1]],
            sss[(_SNCH - 1) % _NBUF], add=True)
        sd[_SNCH - 2].wait()
        sd[_SNCH - 1].wait()

        plsc.subcore_barrier()
        pltpu.sync_copy(acc.at[pl.ds(base, _ROWS_PER_TILE)],
                        p_hbm.at[c].at[pl.ds(base, _ROWS_PER_TILE)])

    return k(u, src3, dst3)


def _sc_degree(dst3):
    @functools.partial(
        pl.kernel,
        out_type=jax.ShapeDtypeStruct((_NC, _NP, _D), jnp.float32),
        mesh=_mesh,
        scratch_types=[
            pltpu.VMEM((_NCH, _CH), jnp.int32),
            pltpu.VMEM((_CH, _D), jnp.float32),
            pltpu.VMEM_SHARED((_NP, _D), jnp.float32),
            pltpu.SemaphoreType.DMA,
        ],
    )
    def k(dst_hbm, deg_hbm, di, ones, acc, sem):
        c = lax.axis_index("c")
        s = lax.axis_index("s")
        w = c * _NS + s
        pltpu.sync_copy(dst_hbm.at[w], di)
        base = s * _ROWS_PER_TILE
        _zero_acc(ones, _CH, _D, acc, base)
        _fill(ones, _CH, _D, 1.0)
        plsc.subcore_barrier()

        @pl.loop(0, _NCH)
        def _(t):
            pltpu.sync_copy(ones, acc.at[di.at[t]], add=True)

        plsc.subcore_barrier()
        pltpu.sync_copy(acc.at[pl.ds(base, _ROWS_PER_TILE)],
                        deg_hbm.at[c].at[pl.ds(base, _ROWS_PER_TILE)])

    return k(dst3)


_R = 400
_G = _N // _R


def _tc_matmul(x, w):
    def body(x_ref, w_ref, acc_ref):
        acc_ref[...] = jnp.dot(x_ref[...], w_ref[...],
                               preferred_element_type=jnp.float32)

    return pl.pallas_call(
        body,
        grid=(_G,),
        in_specs=[
            pl.BlockSpec((_R, _D), lambda i: (i, 0)),
            pl.BlockSpec((_D, _D), lambda i: (0, 0)),
        ],
        out_specs=pl.BlockSpec((_R, _D), lambda i: (i, 0)),
        out_shape=jax.ShapeDtypeStruct((_N, _D), jnp.float32),
    )(x, w)


def _tc_prep(x, degp):
    def body(x_ref, deg_ref, u_ref, dis_ref):
        deg = deg_ref[0][:, 0:1] + deg_ref[1][:, 0:1]
        dis = jnp.where(deg > 0, lax.rsqrt(jnp.maximum(deg, 1e-12)), 0.0)
        dis_ref[...] = dis
        u_ref[...] = x_ref[...] * dis

    return pl.pallas_call(
        body,
        grid=(_G,),
        in_specs=[
            pl.BlockSpec((_R, _D), lambda i: (i, 0)),
            pl.BlockSpec((2, _R, _D), lambda i: (0, i, 0)),
        ],
        out_specs=[
            pl.BlockSpec((_R, _D), lambda i: (i, 0)),
            pl.BlockSpec((_R, 1), lambda i: (i, 0)),
        ],
        out_shape=[
            jax.ShapeDtypeStruct((_N, _D), jnp.float32),
            jax.ShapeDtypeStruct((_N, 1), jnp.float32),
        ],
    )(x, degp)


def _tc_u(p2, dis):
    def body(p_ref, dis_ref, u_ref):
        d = dis_ref[...]
        u_ref[...] = (p_ref[0] + p_ref[1]) * (d * d)

    return pl.pallas_call(
        body,
        grid=(_G,),
        in_specs=[
            pl.BlockSpec((2, _R, _D), lambda i: (0, i, 0)),
            pl.BlockSpec((_R, 1), lambda i: (i, 0)),
        ],
        out_specs=pl.BlockSpec((_R, _D), lambda i: (i, 0)),
        out_shape=jax.ShapeDtypeStruct((_N, _D), jnp.float32),
    )(p2, dis)


def _tc_acc(p2, dis, w, acc):
    def body(p_ref, dis_ref, w_ref, acc_ref, out_ref):
        d = dis_ref[...]
        h = (p_ref[0] + p_ref[1]) * d
        out_ref[...] = acc_ref[...] + jnp.dot(
            h, w_ref[...], preferred_element_type=jnp.float32)

    return pl.pallas_call(
        body,
        grid=(_G,),
        in_specs=[
            pl.BlockSpec((2, _R, _D), lambda i: (0, i, 0)),
            pl.BlockSpec((_R, 1), lambda i: (i, 0)),
            pl.BlockSpec((_D, _D), lambda i: (0, 0)),
            pl.BlockSpec((_R, _D), lambda i: (i, 0)),
        ],
        out_specs=pl.BlockSpec((_R, _D), lambda i: (i, 0)),
        out_shape=jax.ShapeDtypeStruct((_N, _D), jnp.float32),
    )(p2, dis, w, acc)


def _tc_mid(p2, dis, w3, acc, b0, w10):
    def body(p_ref, dis_ref, w3_ref, acc_ref, b_ref, w10_ref,
             acc1_ref, u_ref):
        d = dis_ref[...]
        h = (p_ref[0] + p_ref[1]) * d
        o = (acc_ref[...]
             + jnp.dot(h, w3_ref[...], preferred_element_type=jnp.float32)
             + b_ref[...])
        t = jnp.maximum(o, 0.0)
        acc1_ref[...] = jnp.dot(t, w10_ref[...],
                                preferred_element_type=jnp.float32)
        u_ref[...] = t * d

    return pl.pallas_call(
        body,
        grid=(_G,),
        in_specs=[
            pl.BlockSpec((2, _R, _D), lambda i: (0, i, 0)),
            pl.BlockSpec((_R, 1), lambda i: (i, 0)),
            pl.BlockSpec((_D, _D), lambda i: (0, 0)),
            pl.BlockSpec((_R, _D), lambda i: (i, 0)),
            pl.BlockSpec((1, _D), lambda i: (0, 0)),
            pl.BlockSpec((_D, _D), lambda i: (0, 0)),
        ],
        out_specs=[
            pl.BlockSpec((_R, _D), lambda i: (i, 0)),
            pl.BlockSpec((_R, _D), lambda i: (i, 0)),
        ],
        out_shape=[
            jax.ShapeDtypeStruct((_N, _D), jnp.float32),
            jax.ShapeDtypeStruct((_N, _D), jnp.float32),
        ],
    )(p2, dis, w3, acc, b0, w10)


def _tc_last(p2, dis, w, acc, b1):
    def body(p_ref, dis_ref, w_ref, acc_ref, b_ref, out_ref):
        d = dis_ref[...]
        h = (p_ref[0] + p_ref[1]) * d
        out_ref[...] = (acc_ref[...]
                        + jnp.dot(h, w_ref[...],
                                  preferred_element_type=jnp.float32)
                        + b_ref[...])

    return pl.pallas_call(
        body,
        grid=(_G,),
        in_specs=[
            pl.BlockSpec((2, _R, _D), lambda i: (0, i, 0)),
            pl.BlockSpec((_R, 1), lambda i: (i, 0)),
            pl.BlockSpec((_D, _D), lambda i: (0, 0)),
            pl.BlockSpec((_R, _D), lambda i: (i, 0)),
            pl.BlockSpec((1, _D), lambda i: (0, 0)),
        ],
        out_specs=pl.BlockSpec((_R, _D), lambda i: (i, 0)),
        out_shape=jax.ShapeDtypeStruct((_N, _D), jnp.float32),
    )(p2, dis, w, acc, b1)


def kernel(x, edge_index, W0, b0, W1, b1):
    src = edge_index[0]
    dst = edge_index[1]
    npad = _EP - _E
    pad_src = (jnp.arange(npad, dtype=jnp.int32) * 131) % _N
    pad_dst = _N + (jnp.arange(npad, dtype=jnp.int32) % (_NP - _N))
    srcp = jnp.concatenate([src, pad_src])
    dstp = jnp.concatenate([dst, pad_dst])
    src3 = srcp.reshape(_NC * _NS, _SNCH // _SEC, _SEC, _SCH)
    dst3 = dstp.reshape(_NC * _NS, _SNCH // _SEC, _SEC, _SCH)
    dst3d = dstp.reshape(_NC * _NS, _NCH, _CH)

    degp = _sc_degree(dst3d)
    acc = _tc_matmul(x, W0[0])
    u, dis = _tc_prep(x, degp)
    for k in (1, 2):
        p = _sc_segsum(u, src3, dst3)
        u = _tc_u(p, dis)
        acc = _tc_acc(p, dis, W0[k], acc)
    p = _sc_segsum(u, src3, dst3)
    acc1, u = _tc_mid(p, dis, W0[3], acc, b0.reshape(1, _D), W1[0])
    for k in (1, 2):
        p = _sc_segsum(u, src3, dst3)
        u = _tc_u(p, dis)
        acc1 = _tc_acc(p, dis, W1[k], acc1)
    p = _sc_segsum(u, src3, dst3)
    return _tc_last(p, dis, W1[3], acc1, b1.reshape(1, _D))


# speedup vs baseline: 1.2058x; 1.0025x over previous
import functools

import jax
import jax.numpy as jnp
from jax import lax
from jax.experimental import pallas as pl
from jax.experimental.pallas import tpu as pltpu
from jax.experimental.pallas import tpu_sc as plsc

_N = 10000
_NP = 10240
_E = 320000
_D = 128
_NC = 2
_NS = 16
_CH = 128
_NCH = 80
_SCH = 128
_SNCH = 80
_SEC = 10
_NBUF = 2
_EP = _NC * _NS * _NCH * _CH
_ROWS_PER_TILE = _NP // _NS

_mesh = plsc.VectorSubcoreMesh(core_axis_name="c", subcore_axis_name="s")


def _fill(buf, rows, cols, value):
    @pl.loop(0, rows)
    def _(r):
        for j in range(cols // 16):
            buf.at[pl.ds(r, 1), pl.ds(j * 16, 16)][...] = jnp.full(
                (1, 16), value, jnp.float32)


def _zero_acc(zbuf, rows, cols, acc, base):
    _fill(zbuf, rows, cols, 0.0)
    for q in range(_ROWS_PER_TILE // rows):
        pltpu.sync_copy(zbuf, acc.at[pl.ds(base + q * rows, rows)])


def _sc_segsum(u, src3, dst3):
    @functools.partial(
        pl.kernel,
        out_type=jax.ShapeDtypeStruct((_NC, _NP, _D), jnp.float32),
        mesh=_mesh,
        scratch_types=[
            pltpu.VMEM((2, _SEC, _SCH), jnp.int32),
            pltpu.VMEM((2, _SEC, _SCH), jnp.int32),
            pltpu.VMEM((_NBUF, _SCH, _D), jnp.float32),
            pltpu.VMEM_SHARED((_NP, _D), jnp.float32),
            pltpu.SemaphoreType.DMA,
            pltpu.SemaphoreType.DMA,
            pltpu.SemaphoreType.DMA,
            pltpu.SemaphoreType.DMA,
            pltpu.SemaphoreType.DMA,
            pltpu.SemaphoreType.DMA,
        ],
    )
    def k(u_hbm, src_hbm, dst_hbm, p_hbm, si, di, ring, acc,
          sg0, sg1, ss0, ss1, sxs, sxd):
        sgs = (sg0, sg1)
        sss = (ss0, ss1)
        c = lax.axis_index("c")
        s = lax.axis_index("s")
        w = c * _NS + s
        base = s * _ROWS_PER_TILE
        nsec = _SNCH // _SEC
        ld0 = pltpu.async_copy(src_hbm.at[w, 0], si.at[0], sxs)
        ld1 = pltpu.async_copy(dst_hbm.at[w, 0], di.at[0], sxd)
        _zero_acc(ring.at[0], _SCH, _D, acc, base)
        ld0.wait()
        ld1.wait()
        plsc.subcore_barrier()

        gd = [None] * _SNCH
        sd = [None] * _SNCH
        for t in range(_SNCH):
            q, r = divmod(t, _SEC)
            b = t % _NBUF
            if r == 2 and q + 1 < nsec:
                qb = (q + 1) % 2
                pltpu.async_copy(src_hbm.at[w, q + 1], si.at[qb], sxs)
                pltpu.async_copy(dst_hbm.at[w, q + 1], di.at[qb], sxd)
            if r == 0 and q > 0:
                pltpu.make_async_copy(src_hbm.at[w, 0], si.at[q % 2],
                                      sxs).wait()
                pltpu.make_async_copy(dst_hbm.at[w, 0], di.at[q % 2],
                                      sxd).wait()
            if t >= _NBUF:
                sd[t - _NBUF].wait()
            gd[t] = pltpu.async_copy(u_hbm.at[si.at[q % 2, r]], ring.at[b],
                                     sgs[b])
            if t >= 1:
                t1 = t - 1
                q1, r1 = divmod(t1, _SEC)
                gd[t1].wait()
                sd[t1] = pltpu.async_copy(ring.at[t1 % _NBUF],
                                          acc.at[di.at[q1 % 2, r1]],
                                          sss[t1 % _NBUF], add=True)
        gd[_SNCH - 1].wait()
        sd[_SNCH - 1] = pltpu.async_copy(
            ring.at[(_SNCH - 1) % _NBUF],
            acc.at[di.at[(nsec - 1) % 2, _SEC - 1]],
            sss[(_SNCH - 1) % _NBUF], add=True)
        sd[_SNCH - 2].wait()
        sd[_SNCH - 1].wait()

        plsc.subcore_barrier()
        pltpu.sync_copy(acc.at[pl.ds(base, _ROWS_PER_TILE)],
                        p_hbm.at[c].at[pl.ds(base, _ROWS_PER_TILE)])

    return k(u, src3, dst3)


def _sc_degree(dst3):
    @functools.partial(
        pl.kernel,
        out_type=jax.ShapeDtypeStruct((_NC, _NP, _D), jnp.float32),
        mesh=_mesh,
        scratch_types=[
            pltpu.VMEM((_NCH, _CH), jnp.int32),
            pltpu.VMEM((_CH, _D), jnp.float32),
            pltpu.VMEM_SHARED((_NP, _D), jnp.float32),
            pltpu.SemaphoreType.DMA,
            pltpu.SemaphoreType.DMA,
        ],
    )
    def k(dst_hbm, deg_hbm, di, ones, acc, sem0, sem1):
        sems = (sem0, sem1)
        c = lax.axis_index("c")
        s = lax.axis_index("s")
        w = c * _NS + s
        pltpu.sync_copy(dst_hbm.at[w], di)
        base = s * _ROWS_PER_TILE
        _zero_acc(ones, _CH, _D, acc, base)
        _fill(ones, _CH, _D, 1.0)
        plsc.subcore_barrier()

        @pl.loop(0, _NCH, step=2)
        def _(t0):
            for b in range(2):
                t = t0 + b

                @pl.when(t >= 2)
                def _():
                    pltpu.make_async_copy(ones, acc.at[di.at[0]],
                                          sems[b]).wait()

                pltpu.async_copy(ones, acc.at[di.at[t]], sems[b], add=True)

        for b in range(2):
            pltpu.make_async_copy(ones, acc.at[di.at[0]], sems[b]).wait()

        plsc.subcore_barrier()
        pltpu.sync_copy(acc.at[pl.ds(base, _ROWS_PER_TILE)],
                        deg_hbm.at[c].at[pl.ds(base, _ROWS_PER_TILE)])

    return k(dst3)


_R = 400
_G = _N // _R


def _tc_matmul(x, w):
    def body(x_ref, w_ref, acc_ref):
        acc_ref[...] = jnp.dot(x_ref[...], w_ref[...],
                               preferred_element_type=jnp.float32)

    return pl.pallas_call(
        body,
        grid=(_G,),
        in_specs=[
            pl.BlockSpec((_R, _D), lambda i: (i, 0)),
            pl.BlockSpec((_D, _D), lambda i: (0, 0)),
        ],
        out_specs=pl.BlockSpec((_R, _D), lambda i: (i, 0)),
        out_shape=jax.ShapeDtypeStruct((_N, _D), jnp.float32),
    )(x, w)


def _tc_prep(x, degp):
    def body(x_ref, deg_ref, u_ref, dis_ref):
        deg = deg_ref[0][:, 0:1] + deg_ref[1][:, 0:1]
        dis = jnp.where(deg > 0, lax.rsqrt(jnp.maximum(deg, 1e-12)), 0.0)
        dis_ref[...] = dis
        u_ref[...] = x_ref[...] * dis

    return pl.pallas_call(
        body,
        grid=(_G,),
        in_specs=[
            pl.BlockSpec((_R, _D), lambda i: (i, 0)),
            pl.BlockSpec((2, _R, _D), lambda i: (0, i, 0)),
        ],
        out_specs=[
            pl.BlockSpec((_R, _D), lambda i: (i, 0)),
            pl.BlockSpec((_R, 1), lambda i: (i, 0)),
        ],
        out_shape=[
            jax.ShapeDtypeStruct((_N, _D), jnp.float32),
            jax.ShapeDtypeStruct((_N, 1), jnp.float32),
        ],
    )(x, degp)


def _tc_u(p2, dis):
    def body(p_ref, dis_ref, u_ref):
        d = dis_ref[...]
        u_ref[...] = (p_ref[0] + p_ref[1]) * (d * d)

    return pl.pallas_call(
        body,
        grid=(_G,),
        in_specs=[
            pl.BlockSpec((2, _R, _D), lambda i: (0, i, 0)),
            pl.BlockSpec((_R, 1), lambda i: (i, 0)),
        ],
        out_specs=pl.BlockSpec((_R, _D), lambda i: (i, 0)),
        out_shape=jax.ShapeDtypeStruct((_N, _D), jnp.float32),
    )(p2, dis)


def _tc_acc(p2, dis, w, acc):
    def body(p_ref, dis_ref, w_ref, acc_ref, out_ref):
        d = dis_ref[...]
        h = (p_ref[0] + p_ref[1]) * d
        out_ref[...] = acc_ref[...] + jnp.dot(
            h, w_ref[...], preferred_element_type=jnp.float32)

    return pl.pallas_call(
        body,
        grid=(_G,),
        in_specs=[
            pl.BlockSpec((2, _R, _D), lambda i: (0, i, 0)),
            pl.BlockSpec((_R, 1), lambda i: (i, 0)),
            pl.BlockSpec((_D, _D), lambda i: (0, 0)),
            pl.BlockSpec((_R, _D), lambda i: (i, 0)),
        ],
        out_specs=pl.BlockSpec((_R, _D), lambda i: (i, 0)),
        out_shape=jax.ShapeDtypeStruct((_N, _D), jnp.float32),
    )(p2, dis, w, acc)


def _tc_mid(p2, dis, w3, acc, b0, w10):
    def body(p_ref, dis_ref, w3_ref, acc_ref, b_ref, w10_ref,
             acc1_ref, u_ref):
        d = dis_ref[...]
        h = (p_ref[0] + p_ref[1]) * d
        o = (acc_ref[...]
             + jnp.dot(h, w3_ref[...], preferred_element_type=jnp.float32)
             + b_ref[...])
        t = jnp.maximum(o, 0.0)
        acc1_ref[...] = jnp.dot(t, w10_ref[...],
                                preferred_element_type=jnp.float32)
        u_ref[...] = t * d

    return pl.pallas_call(
        body,
        grid=(_G,),
        in_specs=[
            pl.BlockSpec((2, _R, _D), lambda i: (0, i, 0)),
            pl.BlockSpec((_R, 1), lambda i: (i, 0)),
            pl.BlockSpec((_D, _D), lambda i: (0, 0)),
            pl.BlockSpec((_R, _D), lambda i: (i, 0)),
            pl.BlockSpec((1, _D), lambda i: (0, 0)),
            pl.BlockSpec((_D, _D), lambda i: (0, 0)),
        ],
        out_specs=[
            pl.BlockSpec((_R, _D), lambda i: (i, 0)),
            pl.BlockSpec((_R, _D), lambda i: (i, 0)),
        ],
        out_shape=[
            jax.ShapeDtypeStruct((_N, _D), jnp.float32),
            jax.ShapeDtypeStruct((_N, _D), jnp.float32),
        ],
    )(p2, dis, w3, acc, b0, w10)


def _tc_last(p2, dis, w, acc, b1):
    def body(p_ref, dis_ref, w_ref, acc_ref, b_ref, out_ref):
        d = dis_ref[...]
        h = (p_ref[0] + p_ref[1]) * d
        out_ref[...] = (acc_ref[...]
                        + jnp.dot(h, w_ref[...],
                                  preferred_element_type=jnp.float32)
                        + b_ref[...])

    return pl.pallas_call(
        body,
        grid=(_G,),
        in_specs=[
            pl.BlockSpec((2, _R, _D), lambda i: (0, i, 0)),
            pl.BlockSpec((_R, 1), lambda i: (i, 0)),
            pl.BlockSpec((_D, _D), lambda i: (0, 0)),
            pl.BlockSpec((_R, _D), lambda i: (i, 0)),
            pl.BlockSpec((1, _D), lambda i: (0, 0)),
        ],
        out_specs=pl.BlockSpec((_R, _D), lambda i: (i, 0)),
        out_shape=jax.ShapeDtypeStruct((_N, _D), jnp.float32),
    )(p2, dis, w, acc, b1)


def kernel(x, edge_index, W0, b0, W1, b1):
    src = edge_index[0]
    dst = edge_index[1]
    npad = _EP - _E
    pad_src = (jnp.arange(npad, dtype=jnp.int32) * 131) % _N
    pad_dst = _N + (jnp.arange(npad, dtype=jnp.int32) % (_NP - _N))
    srcp = jnp.concatenate([src, pad_src])
    dstp = jnp.concatenate([dst, pad_dst])
    src3 = srcp.reshape(_NC * _NS, _SNCH // _SEC, _SEC, _SCH)
    dst3 = dstp.reshape(_NC * _NS, _SNCH // _SEC, _SEC, _SCH)
    dst3d = dstp.reshape(_NC * _NS, _NCH, _CH)

    degp = _sc_degree(dst3d)
    acc = _tc_matmul(x, W0[0])
    u, dis = _tc_prep(x, degp)
    for k in (1, 2):
        p = _sc_segsum(u, src3, dst3)
        u = _tc_u(p, dis)
        acc = _tc_acc(p, dis, W0[k], acc)
    p = _sc_segsum(u, src3, dst3)
    acc1, u = _tc_mid(p, dis, W0[3], acc, b0.reshape(1, _D), W1[0])
    for k in (1, 2):
        p = _sc_segsum(u, src3, dst3)
        u = _tc_u(p, dis)
        acc1 = _tc_acc(p, dis, W1[k], acc1)
    p = _sc_segsum(u, src3, dst3)
    return _tc_last(p, dis, W1[3], acc1, b1.reshape(1, _D))
